# baseline (speedup 1.0000x reference)
import functools

import jax
import jax.numpy as jnp
from jax import lax
from jax.experimental import pallas as pl
from jax.experimental.pallas import tpu as pltpu
from jax.experimental.pallas import tpu_sc as plsc

DIM = 64
INP = 128
NTYPE = 3
TYPE_RANGE = 100000


@functools.lru_cache(maxsize=None)
def _make_sc_gather(B, D):
    info = plsc.get_sparse_core_info()
    NC, NS = info.num_cores, info.num_subcores
    NW = NC * NS
    bpw = B // NW
    CH = 128
    NCHUNK = bpw // CH
    mesh = plsc.VectorSubcoreMesh(core_axis_name="c", subcore_axis_name="s")

    @functools.partial(
        pl.kernel, mesh=mesh,
        out_type=jax.ShapeDtypeStruct((B, D), jnp.float32),
        scratch_types=[
            pltpu.VMEM((NCHUNK, CH), jnp.int32),
            pltpu.VMEM((bpw, D), jnp.float32),
            pltpu.SemaphoreType.DMA,
        ],
    )
    def gather(x_hbm, table_hbm, out_hbm, idx_v, rows_v, sem):
        wid = lax.axis_index("s") * NC + lax.axis_index("c")
        base = wid * bpw
        pltpu.sync_copy(x_hbm.at[wid], idx_v)
        for j in range(NCHUNK):
            for l in range(CH // 16):
                v = idx_v[j, pl.ds(l * 16, 16)]
                idx_v[j, pl.ds(l * 16, 16)] = jnp.maximum(v - 1, 0)
        copies = [
            pltpu.async_copy(
                table_hbm.at[idx_v.at[j]],
                rows_v.at[pl.ds(j * CH, CH)],
                sem,
            )
            for j in range(NCHUNK)
        ]
        for c in copies:
            c.wait()
        pltpu.sync_copy(rows_v, out_hbm.at[pl.ds(base, bpw)])

    return gather, NW, NCHUNK, CH


def _tc_body(adj_ref, x_ref, w_ref, b1_ref, b2_ref, g_ref, b_ref,
             out_ref, loss_ref):
    i = pl.program_id(0)
    adj = adj_ref[...]
    xv = x_ref[...]
    valid = xv >= 1
    t = jnp.where(valid, (xv - 1) // TYPE_RANGE, -1)
    Bb = adj.shape[0]

    def sel3(rows):
        return jnp.where(t == 0, rows[0][None, :],
                         jnp.where(t == 1, rows[1][None, :],
                                   rows[2][None, :]))

    wcat = w_ref[...].reshape(NTYPE * DIM, INP)
    h = lax.dot_general(adj, wcat, (((1,), (1,)), ((), ())),
                        precision=lax.Precision.HIGHEST,
                        preferred_element_type=jnp.float32)
    hsel = jnp.where(t == 0, h[:, 0:DIM],
                     jnp.where(t == 1, h[:, DIM:2 * DIM], h[:, 2 * DIM:]))
    enc = jnp.tanh(hsel + sel3(b1_ref))

    z = jnp.zeros((Bb, DIM), jnp.float32)
    encm = jnp.concatenate(
        [jnp.where(t == 0, enc, z),
         jnp.where(t == 1, enc, z),
         jnp.where(t == 2, enc, z)], axis=1)
    recon = lax.dot_general(encm, wcat, (((1,), (0,)), ((), ())),
                            precision=lax.Precision.HIGHEST,
                            preferred_element_type=jnp.float32) \
        + sel3(b2_ref)

    mu = jnp.mean(enc, axis=1, keepdims=True)
    var = jnp.maximum(jnp.mean(enc * enc, axis=1, keepdims=True) - mu * mu,
                      0.0)
    ln = (enc - mu) / jnp.sqrt(var + 1e-5) * sel3(g_ref) + sel3(b_ref)
    out_ref[...] = jnp.where(valid, ln, 0.0)

    ne0 = (adj != 0.0).astype(jnp.float32)
    den = jnp.maximum(jnp.sum(ne0, axis=1, keepdims=True), 1.0)
    num = jnp.sum(ne0 * (adj - recon) ** 2, axis=1, keepdims=True)
    per_row = jnp.where(valid, num / den, 0.0)

    lane = lax.broadcasted_iota(jnp.int32, (Bb, INP), 1)
    onehot = (lane == t).astype(jnp.float32)
    s_blk = jnp.sum(onehot * per_row, axis=0, keepdims=True)
    c_blk = jnp.sum(onehot, axis=0, keepdims=True)

    @pl.when(i == 0)
    def _init():
        loss_ref[...] = jnp.zeros_like(loss_ref)

    loss_ref[0:1, :] += s_blk
    loss_ref[1:2, :] += c_blk

    @pl.when(i == pl.num_programs(0) - 1)
    def _finish():
        s = loss_ref[0:1, :]
        c = loss_ref[1:2, :]
        contrib = jnp.where(c > 0, s / jnp.maximum(c, 1.0), 0.0)
        loss_ref[...] = jnp.broadcast_to(jnp.sum(contrib), loss_ref.shape)


@functools.lru_cache(maxsize=None)
def _make_tc(B, Bb, interpret=False):
    G = B // Bb
    grid_spec = pl.GridSpec(
        grid=(G,),
        in_specs=[
            pl.BlockSpec((Bb, INP), lambda i: (i, 0)),
            pl.BlockSpec((Bb, 1), lambda i: (i, 0)),
            pl.BlockSpec((NTYPE, DIM, INP), lambda i: (0, 0, 0)),
            pl.BlockSpec((NTYPE, DIM), lambda i: (0, 0)),
            pl.BlockSpec((NTYPE, INP), lambda i: (0, 0)),
            pl.BlockSpec((NTYPE, DIM), lambda i: (0, 0)),
            pl.BlockSpec((NTYPE, DIM), lambda i: (0, 0)),
        ],
        out_specs=[
            pl.BlockSpec((Bb, DIM), lambda i: (i, 0)),
            pl.BlockSpec((8, INP), lambda i: (0, 0)),
        ],
    )
    return pl.pallas_call(
        _tc_body,
        grid_spec=grid_spec,
        out_shape=[
            jax.ShapeDtypeStruct((B, DIM), jnp.float32),
            jax.ShapeDtypeStruct((8, INP), jnp.float32),
        ],
        compiler_params=pltpu.CompilerParams(
            dimension_semantics=("arbitrary",)),
        interpret=interpret,
    )


def kernel(x, tables, ae_w, ae_b1, ae_b2, ln_g, ln_b):
    B = x.shape[0]
    x = x.astype(jnp.int32)
    table_flat = tables.reshape(NTYPE * tables.shape[1], INP)

    gather, NW, NCHUNK, CH = _make_sc_gather(B, INP)
    adj = gather(x.reshape(NW, NCHUNK, CH), table_flat)

    tc = _make_tc(B, 2048)
    final, lossbuf = tc(adj, x.reshape(B, 1), ae_w, ae_b1, ae_b2, ln_g, ln_b)
    return final, lossbuf[0, 0].reshape(1)


# speedup vs baseline: 1.0243x; 1.0243x over previous
import functools

import jax
import jax.numpy as jnp
from jax import lax
from jax.experimental import pallas as pl
from jax.experimental.pallas import tpu as pltpu
from jax.experimental.pallas import tpu_sc as plsc

DIM = 64
INP = 128
NTYPE = 3
TYPE_RANGE = 100000


@functools.lru_cache(maxsize=None)
def _make_sc_gather(B, D):
    info = plsc.get_sparse_core_info()
    NC, NS = info.num_cores, info.num_subcores
    NW = NC * NS
    bpw = B // NW
    CH = 128
    NCHUNK = bpw // CH
    mesh = plsc.VectorSubcoreMesh(core_axis_name="c", subcore_axis_name="s")

    @functools.partial(
        pl.kernel, mesh=mesh,
        out_type=jax.ShapeDtypeStruct((B, D), jnp.float32),
        scratch_types=[
            pltpu.VMEM((NCHUNK, CH), jnp.int32),
            pltpu.VMEM((bpw, D), jnp.float32),
            pltpu.SemaphoreType.DMA,
        ],
    )
    def gather(x_hbm, table_hbm, out_hbm, idx_v, rows_v, sem):
        wid = lax.axis_index("s") * NC + lax.axis_index("c")
        base = wid * bpw
        pltpu.sync_copy(x_hbm.at[wid], idx_v)
        for j in range(NCHUNK):
            for l in range(CH // 16):
                v = idx_v[j, pl.ds(l * 16, 16)]
                idx_v[j, pl.ds(l * 16, 16)] = jnp.maximum(v - 1, 0)
        copies = [
            pltpu.async_copy(
                table_hbm.at[idx_v.at[j]],
                rows_v.at[pl.ds(j * CH, CH)],
                sem,
            )
            for j in range(NCHUNK)
        ]
        for c in copies:
            c.wait()
        pltpu.sync_copy(rows_v, out_hbm.at[pl.ds(base, bpw)])

    return gather, NW, NCHUNK, CH


def _tc_body(adj_ref, x_ref, w_ref, b1_ref, b2_ref, g_ref, b_ref,
             out_ref, loss_ref):
    i = pl.program_id(0)
    adj = adj_ref[...]
    xv = x_ref[...]
    valid = xv >= 1
    t = jnp.where(valid, (xv - 1) // TYPE_RANGE, -1)
    Bb = adj.shape[0]

    def sel3(a0, a1, a2):
        return jnp.where(t == 0, a0, jnp.where(t == 1, a1, a2))

    h = [lax.dot_general(adj, w_ref[k], (((1,), (1,)), ((), ())),
                         precision=lax.Precision.HIGHEST,
                         preferred_element_type=jnp.float32)
         + b1_ref[k][None, :]
         for k in range(NTYPE)]
    enc = jnp.tanh(sel3(*h))

    z = jnp.zeros((Bb, DIM), jnp.float32)
    recon0 = sum(
        lax.dot_general(jnp.where(t == k, enc, z), w_ref[k],
                        (((1,), (0,)), ((), ())),
                        precision=lax.Precision.HIGHEST,
                        preferred_element_type=jnp.float32)
        for k in range(NTYPE))
    amb = sel3(*[adj - b2_ref[k][None, :] for k in range(NTYPE)])

    mu = jnp.mean(enc, axis=1, keepdims=True)
    var = jnp.maximum(jnp.mean(enc * enc, axis=1, keepdims=True) - mu * mu,
                      0.0)
    nrm = (enc - mu) * lax.rsqrt(var + 1e-5)
    ln = sel3(*[nrm * g_ref[k][None, :] + b_ref[k][None, :]
                for k in range(NTYPE)])
    out_ref[...] = jnp.where(valid, ln, 0.0)

    ne0 = (adj != 0.0).astype(jnp.float32)
    den = jnp.maximum(jnp.sum(ne0, axis=1, keepdims=True), 1.0)
    num = jnp.sum(ne0 * (amb - recon0) ** 2, axis=1, keepdims=True)
    per_row = jnp.where(valid, num / den, 0.0)

    lane = lax.broadcasted_iota(jnp.int32, (Bb, INP), 1)
    onehot = (lane == t).astype(jnp.float32)
    s_blk = jnp.sum(onehot * per_row, axis=0, keepdims=True)
    c_blk = jnp.sum(onehot, axis=0, keepdims=True)

    @pl.when(i == 0)
    def _init():
        loss_ref[...] = jnp.zeros_like(loss_ref)

    loss_ref[0:1, :] += s_blk
    loss_ref[1:2, :] += c_blk

    @pl.when(i == pl.num_programs(0) - 1)
    def _finish():
        s = loss_ref[0:1, :]
        c = loss_ref[1:2, :]
        contrib = jnp.where(c > 0, s / jnp.maximum(c, 1.0), 0.0)
        loss_ref[...] = jnp.broadcast_to(jnp.sum(contrib), loss_ref.shape)


@functools.lru_cache(maxsize=None)
def _make_tc(B, Bb, interpret=False):
    G = B // Bb
    grid_spec = pl.GridSpec(
        grid=(G,),
        in_specs=[
            pl.BlockSpec((Bb, INP), lambda i: (i, 0)),
            pl.BlockSpec((Bb, 1), lambda i: (i, 0)),
            pl.BlockSpec((NTYPE, DIM, INP), lambda i: (0, 0, 0)),
            pl.BlockSpec((NTYPE, DIM), lambda i: (0, 0)),
            pl.BlockSpec((NTYPE, INP), lambda i: (0, 0)),
            pl.BlockSpec((NTYPE, DIM), lambda i: (0, 0)),
            pl.BlockSpec((NTYPE, DIM), lambda i: (0, 0)),
        ],
        out_specs=[
            pl.BlockSpec((Bb, DIM), lambda i: (i, 0)),
            pl.BlockSpec((8, INP), lambda i: (0, 0)),
        ],
    )
    return pl.pallas_call(
        _tc_body,
        grid_spec=grid_spec,
        out_shape=[
            jax.ShapeDtypeStruct((B, DIM), jnp.float32),
            jax.ShapeDtypeStruct((8, INP), jnp.float32),
        ],
        compiler_params=pltpu.CompilerParams(
            dimension_semantics=("arbitrary",)),
        interpret=interpret,
    )


def kernel(x, tables, ae_w, ae_b1, ae_b2, ln_g, ln_b):
    B = x.shape[0]
    x = x.astype(jnp.int32)
    table_flat = tables.reshape(NTYPE * tables.shape[1], INP)

    gather, NW, NCHUNK, CH = _make_sc_gather(B, INP)
    adj = gather(x.reshape(NW, NCHUNK, CH), table_flat)

    tc = _make_tc(B, 2048)
    final, lossbuf = tc(adj, x.reshape(B, 1), ae_w, ae_b1, ae_b2, ln_g, ln_b)
    return final, lossbuf[0, 0].reshape(1)


# speedup vs baseline: 2.0571x; 2.0082x over previous
import functools

import jax
import jax.numpy as jnp
from jax import lax
from jax.experimental import pallas as pl
from jax.experimental.pallas import tpu as pltpu
from jax.experimental.pallas import tpu_sc as plsc

DIM = 64
INP = 128
NTYPE = 3
TYPE_RANGE = 100000


@functools.lru_cache(maxsize=None)
def _make_sc_gather(B, D):
    info = plsc.get_sparse_core_info()
    NC, NS = info.num_cores, info.num_subcores
    NW = NC * NS
    bpw = B // NW
    CH = 128
    NCHUNK = bpw // CH
    mesh = plsc.VectorSubcoreMesh(core_axis_name="c", subcore_axis_name="s")

    @functools.partial(
        pl.kernel, mesh=mesh,
        out_type=jax.ShapeDtypeStruct((B, D), jnp.float32),
        scratch_types=[
            pltpu.VMEM((NCHUNK, CH), jnp.int32),
            pltpu.VMEM((bpw, D), jnp.float32),
            pltpu.SemaphoreType.DMA,
        ],
    )
    def gather(x_hbm, table_hbm, out_hbm, idx_v, rows_v, sem):
        wid = lax.axis_index("s") * NC + lax.axis_index("c")
        base = wid * bpw
        pltpu.sync_copy(x_hbm.at[wid], idx_v)
        for j in range(NCHUNK):
            for l in range(CH // 16):
                v = idx_v[j, pl.ds(l * 16, 16)]
                idx_v[j, pl.ds(l * 16, 16)] = jnp.maximum(v - 1, 0)
        copies = [
            pltpu.async_copy(
                table_hbm.at[idx_v.at[j]],
                rows_v.at[pl.ds(j * CH, CH)],
                sem,
            )
            for j in range(NCHUNK)
        ]
        for c in copies:
            c.wait()
        pltpu.sync_copy(rows_v, out_hbm.at[pl.ds(base, bpw)])

    return gather, NW, NCHUNK, CH


def _tc_body(adj_ref, x_ref, w_ref, b1_ref, b2_ref, g_ref, b_ref,
             out_ref, loss_ref):
    i = pl.program_id(0)
    adj = adj_ref[...]
    xv = x_ref[...]
    valid = xv >= 1
    t = jnp.where(valid, (xv - 1) // TYPE_RANGE, -1)
    Bb = adj.shape[0]

    def sel3(a0, a1, a2):
        return jnp.where(t == 0, a0, jnp.where(t == 1, a1, a2))

    h = [lax.dot_general(adj, w_ref[k], (((1,), (1,)), ((), ())),
                         preferred_element_type=jnp.float32)
         + b1_ref[k][None, :]
         for k in range(NTYPE)]
    enc = jnp.tanh(sel3(*h))

    z = jnp.zeros((Bb, DIM), jnp.float32)
    recon0 = sum(
        lax.dot_general(jnp.where(t == k, enc, z), w_ref[k],
                        (((1,), (0,)), ((), ())),
                        preferred_element_type=jnp.float32)
        for k in range(NTYPE))
    amb = sel3(*[adj - b2_ref[k][None, :] for k in range(NTYPE)])

    mu = jnp.mean(enc, axis=1, keepdims=True)
    var = jnp.maximum(jnp.mean(enc * enc, axis=1, keepdims=True) - mu * mu,
                      0.0)
    nrm = (enc - mu) * lax.rsqrt(var + 1e-5)
    ln = sel3(*[nrm * g_ref[k][None, :] + b_ref[k][None, :]
                for k in range(NTYPE)])
    out_ref[...] = jnp.where(valid, ln, 0.0)

    ne0 = (adj != 0.0).astype(jnp.float32)
    den = jnp.maximum(jnp.sum(ne0, axis=1, keepdims=True), 1.0)
    num = jnp.sum(ne0 * (amb - recon0) ** 2, axis=1, keepdims=True)
    per_row = jnp.where(valid, num / den, 0.0)

    lane = lax.broadcasted_iota(jnp.int32, (Bb, INP), 1)
    onehot = (lane == t).astype(jnp.float32)
    s_blk = jnp.sum(onehot * per_row, axis=0, keepdims=True)
    c_blk = jnp.sum(onehot, axis=0, keepdims=True)

    @pl.when(i == 0)
    def _init():
        loss_ref[...] = jnp.zeros_like(loss_ref)

    loss_ref[0:1, :] += s_blk
    loss_ref[1:2, :] += c_blk

    @pl.when(i == pl.num_programs(0) - 1)
    def _finish():
        s = loss_ref[0:1, :]
        c = loss_ref[1:2, :]
        contrib = jnp.where(c > 0, s / jnp.maximum(c, 1.0), 0.0)
        loss_ref[...] = jnp.broadcast_to(jnp.sum(contrib), loss_ref.shape)


@functools.lru_cache(maxsize=None)
def _make_tc(B, Bb, interpret=False):
    G = B // Bb
    grid_spec = pl.GridSpec(
        grid=(G,),
        in_specs=[
            pl.BlockSpec((Bb, INP), lambda i: (i, 0)),
            pl.BlockSpec((Bb, 1), lambda i: (i, 0)),
            pl.BlockSpec((NTYPE, DIM, INP), lambda i: (0, 0, 0)),
            pl.BlockSpec((NTYPE, DIM), lambda i: (0, 0)),
            pl.BlockSpec((NTYPE, INP), lambda i: (0, 0)),
            pl.BlockSpec((NTYPE, DIM), lambda i: (0, 0)),
            pl.BlockSpec((NTYPE, DIM), lambda i: (0, 0)),
        ],
        out_specs=[
            pl.BlockSpec((Bb, DIM), lambda i: (i, 0)),
            pl.BlockSpec((8, INP), lambda i: (0, 0)),
        ],
    )
    return pl.pallas_call(
        _tc_body,
        grid_spec=grid_spec,
        out_shape=[
            jax.ShapeDtypeStruct((B, DIM), jnp.float32),
            jax.ShapeDtypeStruct((8, INP), jnp.float32),
        ],
        compiler_params=pltpu.CompilerParams(
            dimension_semantics=("arbitrary",)),
        interpret=interpret,
    )


def kernel(x, tables, ae_w, ae_b1, ae_b2, ln_g, ln_b):
    B = x.shape[0]
    x = x.astype(jnp.int32)
    table_flat = tables.reshape(NTYPE * tables.shape[1], INP)

    gather, NW, NCHUNK, CH = _make_sc_gather(B, INP)
    adj = gather(x.reshape(NW, NCHUNK, CH), table_flat)

    tc = _make_tc(B, 2048)
    final, lossbuf = tc(adj, x.reshape(B, 1), ae_w, ae_b1, ae_b2, ln_g, ln_b)
    return final, lossbuf[0, 0].reshape(1)
